# baseline (speedup 1.0000x reference)
import functools

import jax
import jax.numpy as jnp
from jax import lax
from jax.experimental import pallas as pl
from jax.experimental.pallas import tpu as pltpu
from jax.experimental.pallas import tpu_sc as plsc

_B = 8
_S = 512
_D = 512
_MAX_REL = 32
_VOCAB = 2 * _MAX_REL + 1
_NC = 2
_NS = 16
_NW = _NC * _NS
_ROWS = _S // _NW
_L = 16
_NCH = _D // _L
_CHUNK = _ROWS * _D


def _sc_body(x_hbm, pe_hbm, out_hbm, pe_v, pe64_v, rowpe_v, xa, xb,
             sin_a, sin_b, sout_a, sout_b):
    wid = lax.axis_index("s") * _NC + lax.axis_index("c")
    base = wid * _ROWS

    pltpu.sync_copy(pe_hbm, pe_v)

    def _save64(c, carry):
        sl = pl.ds(c * _L, _L)
        pe64_v[0, sl] = pe_v[_VOCAB - 1, sl]
        return carry

    lax.fori_loop(0, _NCH, _save64, 0, unroll=8)

    def _cum_m(m, carry):
        def _cum_c(c, carry2):
            sl = pl.ds(c * _L, _L)
            pe_v[m, sl] = pe_v[m, sl] + pe_v[m - 1, sl]
            return carry2

        return lax.fori_loop(0, _NCH, _cum_c, carry, unroll=8)

    lax.fori_loop(1, _VOCAB, _cum_m, 0)

    inv = jnp.float32(1.0 / _S)

    def _row_t(t, carry):
        i = base + t
        a = jnp.maximum(_S - _MAX_REL - i, 0).astype(jnp.float32)
        b = jnp.maximum(i - (_MAX_REL - 1), 0).astype(jnp.float32)
        hi = jnp.minimum(i + _MAX_REL, _VOCAB - 2)
        lo = jnp.maximum(i - (_S - _MAX_REL), 0)

        def _row_c(c, carry2):
            sl = pl.ds(c * _L, _L)
            val = (
                a * pe_v[0, sl]
                + b * pe64_v[0, sl]
                + pe_v[hi, sl]
                - pe_v[lo, sl]
            ) * inv
            rowpe_v[pl.ds(t * _D + c * _L, _L)] = val
            return carry2

        return lax.fori_loop(0, _NCH, _row_c, carry, unroll=8)

    lax.fori_loop(0, _ROWS, _row_t, 0)

    def _off(b_i):
        return b_i * (_S * _D) + base * _D

    def _add_into(buf):
        def _add_q(q, carry):
            sl = pl.ds(q * _L, _L)
            buf[sl] = buf[sl] + rowpe_v[sl]
            return carry

        lax.fori_loop(0, _CHUNK // _L, _add_q, 0, unroll=8)

    bufs = [(xa, sin_a, sout_a), (xb, sin_b, sout_b)]
    ins = [None] * _B
    outs = [None] * _B
    ins[0] = pltpu.async_copy(x_hbm.at[pl.ds(_off(0), _CHUNK)], xa, sin_a)
    for b_i in range(_B):
        buf, _, sout = bufs[b_i % 2]
        nbuf, nsin, _ = bufs[(b_i + 1) % 2]
        if b_i + 1 < _B:
            if b_i >= 1:
                outs[b_i - 1].wait()
            ins[b_i + 1] = pltpu.async_copy(
                x_hbm.at[pl.ds(_off(b_i + 1), _CHUNK)], nbuf, nsin)
        ins[b_i].wait()
        _add_into(buf)
        outs[b_i] = pltpu.async_copy(
            buf, out_hbm.at[pl.ds(_off(b_i), _CHUNK)], sout)
    outs[_B - 1].wait()


@jax.jit
def kernel(x, pe):
    b, s, d = x.shape
    mesh = plsc.VectorSubcoreMesh(
        core_axis_name="c", subcore_axis_name="s",
        num_cores=_NC, num_subcores=_NS,
    )
    sc_call = pl.kernel(
        _sc_body,
        out_type=jax.ShapeDtypeStruct((b * s * d,), jnp.float32),
        mesh=mesh,
        scratch_types=[
            pltpu.VMEM((_VOCAB, d), jnp.float32),
            pltpu.VMEM((1, d), jnp.float32),
            pltpu.VMEM((_CHUNK,), jnp.float32),
            pltpu.VMEM((_CHUNK,), jnp.float32),
            pltpu.VMEM((_CHUNK,), jnp.float32),
            pltpu.SemaphoreType.DMA,
            pltpu.SemaphoreType.DMA,
            pltpu.SemaphoreType.DMA,
            pltpu.SemaphoreType.DMA,
        ],
    )
    return sc_call(x.reshape(-1), pe).reshape(b, s, d)


# speedup vs baseline: 1.4111x; 1.4111x over previous
import functools

import jax
import jax.numpy as jnp
from jax import lax
from jax.experimental import pallas as pl
from jax.experimental.pallas import tpu as pltpu
from jax.experimental.pallas import tpu_sc as plsc

_B = 8
_S = 512
_D = 512
_MAX_REL = 32
_VOCAB = 2 * _MAX_REL + 1
_NC = 2
_NS = 16
_NW = _NC * _NS
_ROWS = _S // _NW
_L = 16
_NCH = _D // _L
_CHUNK = _ROWS * _D


def _sc_body(x_hbm, pe_hbm, out_hbm, pe_v, pe64_v, rowpe_v, xa, xb,
             sin_a, sin_b, sout_a, sout_b):
    wid = lax.axis_index("s") * _NC + lax.axis_index("c")
    base = wid * _ROWS

    pltpu.sync_copy(pe_hbm, pe_v)

    @plsc.parallel_loop(0, _NCH, unroll=8)
    def _save64(c):
        sl = pl.ds(c * _L, _L)
        pe64_v[0, sl] = pe_v[_VOCAB - 1, sl]

    def _cum_m(m, carry):
        @plsc.parallel_loop(0, _NCH, unroll=8)
        def _cum_c(c):
            sl = pl.ds(c * _L, _L)
            pe_v[m, sl] = pe_v[m, sl] + pe_v[m - 1, sl]

        return carry

    lax.fori_loop(1, _VOCAB, _cum_m, 0)

    inv = jnp.float32(1.0 / _S)

    def _row_t(t, carry):
        i = base + t
        a = jnp.maximum(_S - _MAX_REL - i, 0).astype(jnp.float32)
        b = jnp.maximum(i - (_MAX_REL - 1), 0).astype(jnp.float32)
        hi = jnp.minimum(i + _MAX_REL, _VOCAB - 2)
        lo = jnp.maximum(i - (_S - _MAX_REL), 0)

        @plsc.parallel_loop(0, _NCH, unroll=8)
        def _row_c(c):
            sl = pl.ds(c * _L, _L)
            val = (
                a * pe_v[0, sl]
                + b * pe64_v[0, sl]
                + pe_v[hi, sl]
                - pe_v[lo, sl]
            ) * inv
            rowpe_v[pl.ds(t * _D + c * _L, _L)] = val

        return carry

    lax.fori_loop(0, _ROWS, _row_t, 0)

    def _off(b_i):
        return b_i * (_S * _D) + base * _D

    def _add_into(buf):
        @plsc.parallel_loop(0, _CHUNK // _L, unroll=8)
        def _add_q(q):
            sl = pl.ds(q * _L, _L)
            buf[sl] = buf[sl] + rowpe_v[sl]

    bufs = [(xa, sin_a, sout_a), (xb, sin_b, sout_b)]
    ins = [None] * _B
    outs = [None] * _B
    ins[0] = pltpu.async_copy(x_hbm.at[pl.ds(_off(0), _CHUNK)], xa, sin_a)
    for b_i in range(_B):
        buf, _, sout = bufs[b_i % 2]
        nbuf, nsin, _ = bufs[(b_i + 1) % 2]
        if b_i + 1 < _B:
            if b_i >= 1:
                outs[b_i - 1].wait()
            ins[b_i + 1] = pltpu.async_copy(
                x_hbm.at[pl.ds(_off(b_i + 1), _CHUNK)], nbuf, nsin)
        ins[b_i].wait()
        _add_into(buf)
        outs[b_i] = pltpu.async_copy(
            buf, out_hbm.at[pl.ds(_off(b_i), _CHUNK)], sout)
    outs[_B - 1].wait()


@jax.jit
def kernel(x, pe):
    b, s, d = x.shape
    mesh = plsc.VectorSubcoreMesh(
        core_axis_name="c", subcore_axis_name="s",
        num_cores=_NC, num_subcores=_NS,
    )
    sc_call = pl.kernel(
        _sc_body,
        out_type=jax.ShapeDtypeStruct((b * s * d,), jnp.float32),
        mesh=mesh,
        scratch_types=[
            pltpu.VMEM((_VOCAB, d), jnp.float32),
            pltpu.VMEM((1, d), jnp.float32),
            pltpu.VMEM((_CHUNK,), jnp.float32),
            pltpu.VMEM((_CHUNK,), jnp.float32),
            pltpu.VMEM((_CHUNK,), jnp.float32),
            pltpu.SemaphoreType.DMA,
            pltpu.SemaphoreType.DMA,
            pltpu.SemaphoreType.DMA,
            pltpu.SemaphoreType.DMA,
        ],
    )
    return sc_call(x.reshape(-1), pe).reshape(b, s, d)


# speedup vs baseline: 2.1293x; 1.5089x over previous
import functools

import jax
import jax.numpy as jnp
from jax import lax
from jax.experimental import pallas as pl
from jax.experimental.pallas import tpu as pltpu
from jax.experimental.pallas import tpu_sc as plsc

_B = 8
_S = 512
_D = 512
_MAX_REL = 32
_VOCAB = 2 * _MAX_REL + 1
_NC = 2
_NS = 16
_NW = _NC * _NS
_ROWS = _S // _NW
_L = 16
_NCH = _D // _L


def _sc_body(x_hbm, pe_hbm, out_hbm, pe_v, pe64_v, rowpe_v, xa, xb,
             sin_a, sin_b, sout_a, sout_b):
    wid = lax.axis_index("s") * _NC + lax.axis_index("c")
    base = wid * _ROWS

    pltpu.sync_copy(pe_hbm, pe_v)

    @plsc.parallel_loop(0, _NCH, unroll=8)
    def _save64(c):
        sl = pl.ds(c * _L, _L)
        pe64_v[0, sl] = pe_v[_VOCAB - 1, sl]

    def _cum_m(m, carry):
        @plsc.parallel_loop(0, _NCH, unroll=8)
        def _cum_c(c):
            sl = pl.ds(c * _L, _L)
            pe_v[m, sl] = pe_v[m, sl] + pe_v[m - 1, sl]

        return carry

    lax.fori_loop(1, _VOCAB, _cum_m, 0)

    inv = jnp.float32(1.0 / _S)

    def _row_t(t, carry):
        i = base + t
        a = jnp.maximum(_S - _MAX_REL - i, 0).astype(jnp.float32)
        b = jnp.maximum(i - (_MAX_REL - 1), 0).astype(jnp.float32)
        hi = jnp.minimum(i + _MAX_REL, _VOCAB - 2)
        lo = jnp.maximum(i - (_S - _MAX_REL), 0)

        @plsc.parallel_loop(0, _NCH, unroll=8)
        def _row_c(c):
            sl = pl.ds(c * _L, _L)
            val = (
                a * pe_v[0, sl]
                + b * pe64_v[0, sl]
                + pe_v[hi, sl]
                - pe_v[lo, sl]
            ) * inv
            rowpe_v[t, sl] = val

        return carry

    lax.fori_loop(0, _ROWS, _row_t, 0)

    def _add_into(buf):
        @plsc.parallel_loop(0, _ROWS * _NCH, unroll=8)
        def _add_q(q):
            t = q // _NCH
            sl = pl.ds((q % _NCH) * _L, _L)
            buf[t, sl] = buf[t, sl] + rowpe_v[t, sl]

    bufs = [(xa, sin_a, sout_a), (xb, sin_b, sout_b)]
    ins = [None] * _B
    outs = [None] * _B
    ins[0] = pltpu.async_copy(x_hbm.at[0, pl.ds(base, _ROWS)], xa, sin_a)
    for b_i in range(_B):
        buf, _, sout = bufs[b_i % 2]
        nbuf, nsin, _ = bufs[(b_i + 1) % 2]
        if b_i + 1 < _B:
            if b_i >= 1:
                outs[b_i - 1].wait()
            ins[b_i + 1] = pltpu.async_copy(
                x_hbm.at[b_i + 1, pl.ds(base, _ROWS)], nbuf, nsin)
        ins[b_i].wait()
        _add_into(buf)
        outs[b_i] = pltpu.async_copy(
            buf, out_hbm.at[b_i, pl.ds(base, _ROWS)], sout)
    outs[_B - 1].wait()


@jax.jit
def kernel(x, pe):
    b, s, d = x.shape
    mesh = plsc.VectorSubcoreMesh(
        core_axis_name="c", subcore_axis_name="s",
        num_cores=_NC, num_subcores=_NS,
    )
    sc_call = pl.kernel(
        _sc_body,
        out_type=jax.ShapeDtypeStruct((b, s, d), jnp.float32),
        mesh=mesh,
        scratch_types=[
            pltpu.VMEM((_VOCAB, d), jnp.float32),
            pltpu.VMEM((1, d), jnp.float32),
            pltpu.VMEM((_ROWS, d), jnp.float32),
            pltpu.VMEM((_ROWS, d), jnp.float32),
            pltpu.VMEM((_ROWS, d), jnp.float32),
            pltpu.SemaphoreType.DMA,
            pltpu.SemaphoreType.DMA,
            pltpu.SemaphoreType.DMA,
            pltpu.SemaphoreType.DMA,
        ],
        compiler_params=pltpu.CompilerParams(use_tc_tiling_on_sc=True),
    )
    return sc_call(x, pe)


# speedup vs baseline: 10.3414x; 4.8567x over previous
import functools

import jax
import jax.numpy as jnp
from jax import lax
from jax.experimental import pallas as pl
from jax.experimental.pallas import tpu as pltpu

_S = 512
_D = 512
_MAX_REL = 32
_VOCAB = 2 * _MAX_REL + 1
_KPAD = 128
_BB = 4


def _rpe_kernel(x_ref, pe_ref, out_ref, rowpe_ref):
    b = pl.program_id(0)

    @pl.when(b == 0)
    def _compute_row_pe():
        i = lax.broadcasted_iota(jnp.int32, (_S, _KPAD), 0)
        k = lax.broadcasted_iota(jnp.int32, (_S, _KPAD), 1)
        r = k - _MAX_REL
        mid = ((k >= 1) & (k <= _VOCAB - 2) & (r <= i) & (r >= i - (_S - 1)))
        counts = mid.astype(jnp.float32)
        left = jnp.maximum(_S - _MAX_REL - i, 0).astype(jnp.float32)
        right = jnp.maximum(i - _MAX_REL + 1, 0).astype(jnp.float32)
        counts = counts + jnp.where(k == 0, left, 0.0)
        counts = counts + jnp.where(k == _VOCAB - 1, right, 0.0)
        rowpe_ref[...] = jnp.dot(
            counts.astype(jnp.bfloat16), pe_ref[...],
            preferred_element_type=jnp.float32,
        ) * (1.0 / _S)

    out_ref[...] = x_ref[...] + rowpe_ref[...][None]


@jax.jit
def kernel(x, pe):
    b, s, d = x.shape
    pe_padded = jnp.zeros((_KPAD, d), dtype=pe.dtype).at[: pe.shape[0]].set(pe)
    return pl.pallas_call(
        _rpe_kernel,
        grid=(b // _BB,),
        in_specs=[
            pl.BlockSpec((_BB, s, d), lambda i: (i, 0, 0)),
            pl.BlockSpec((_KPAD, d), lambda i: (0, 0)),
        ],
        out_specs=pl.BlockSpec((_BB, s, d), lambda i: (i, 0, 0)),
        out_shape=jax.ShapeDtypeStruct((b, s, d), x.dtype),
        scratch_shapes=[pltpu.VMEM((s, d), jnp.float32)],
        compiler_params=pltpu.CompilerParams(
            dimension_semantics=("arbitrary",),
        ),
    )(x, pe_padded)


# speedup vs baseline: 10.4112x; 1.0068x over previous
import functools

import jax
import jax.numpy as jnp
from jax import lax
from jax.experimental import pallas as pl
from jax.experimental.pallas import tpu as pltpu

_S = 512
_D = 512
_MAX_REL = 32
_VOCAB = 2 * _MAX_REL + 1
_KPAD = 128
_BB = 4


def _rpe_kernel(x_ref, pe_ref, out_ref, rowpe_ref):
    b = pl.program_id(0)

    @pl.when(b == 0)
    def _compute_row_pe():
        i = lax.broadcasted_iota(jnp.int32, (_S, _KPAD), 0)
        k = lax.broadcasted_iota(jnp.int32, (_S, _KPAD), 1)
        r = k - _MAX_REL
        mid = ((k >= 1) & (k <= _VOCAB - 2) & (r <= i) & (r >= i - (_S - 1)))
        counts = mid.astype(jnp.float32)
        left = jnp.maximum(_S - _MAX_REL - i, 0).astype(jnp.float32)
        right = jnp.maximum(i - _MAX_REL + 1, 0).astype(jnp.float32)
        counts = counts + jnp.where(k == 0, left, 0.0)
        counts = counts + jnp.where(k == _VOCAB - 1, right, 0.0)
        rowpe_ref[...] = jnp.dot(
            counts, pe_ref[...], preferred_element_type=jnp.float32
        ) * (1.0 / _S)

    out_ref[...] = x_ref[...] + rowpe_ref[...][None]


@jax.jit
def kernel(x, pe):
    b, s, d = x.shape
    pe_padded = jnp.zeros((_KPAD, d), dtype=pe.dtype).at[: pe.shape[0]].set(pe)
    return pl.pallas_call(
        _rpe_kernel,
        grid=(b // _BB,),
        in_specs=[
            pl.BlockSpec((_BB, s, d), lambda i: (i, 0, 0)),
            pl.BlockSpec((_KPAD, d), lambda i: (0, 0)),
        ],
        out_specs=pl.BlockSpec((_BB, s, d), lambda i: (i, 0, 0)),
        out_shape=jax.ShapeDtypeStruct((b, s, d), x.dtype),
        scratch_shapes=[pltpu.VMEM((s, d), jnp.float32)],
        compiler_params=pltpu.CompilerParams(
            dimension_semantics=("arbitrary",),
        ),
    )(x, pe_padded)
